# Initial kernel scaffold; baseline (speedup 1.0000x reference)
import jax
import jax.numpy as jnp
from jax.experimental import pallas as pl


def kernel(x, edge_index, edge_weight, batch, W1, b1, Wlin, blin):
    raise NotImplementedError("write your pallas kernel here")


# speedup vs baseline: 10.5832x; 10.5832x over previous
import functools

import jax
import jax.numpy as jnp
from jax import lax
from jax.experimental import pallas as pl
from jax.experimental.pallas import tpu as pltpu
from jax.experimental.pallas import tpu_sc as plsc

N = 10000
DIN = 128
DH = 64
DOUT = 16
G = 32

NC = 2
NS = 16
NW = NC * NS
N_PAD = 10240
ROWS_T = N_PAD // NS

E_RAW = 320000
E_PAD = 330240
ET = E_PAD // NW
C = 80
NCHUNK = ET // C

ZB = 64
F32 = jnp.float32

_mesh = plsc.VectorSubcoreMesh(core_axis_name="c", subcore_axis_name="s")


@functools.partial(
    pl.kernel,
    out_type=jax.ShapeDtypeStruct((NC * N_PAD,), F32),
    mesh=_mesh,
    scratch_types=[
        pltpu.VMEM((C,), jnp.int32),
        pltpu.VMEM((C,), F32),
        pltpu.VMEM((ROWS_T,), F32),
        pltpu.VMEM_SHARED((N_PAD,), F32),
    ],
)
def _deg_kernel(dst_hbm, w_hbm, degp_hbm, didx, wv, stage, deg_sh):
    c = lax.axis_index("c")
    s = lax.axis_index("s")

    for i in range(ROWS_T // 16):
        stage[pl.ds(i * 16, 16)] = jnp.zeros((16,), F32)
    pltpu.sync_copy(stage, deg_sh.at[pl.ds(s * ROWS_T, ROWS_T)])
    plsc.subcore_barrier()

    tile_base = (c * NS + s) * ET

    def body(i, carry):
        base = tile_base + i * C
        pltpu.sync_copy(dst_hbm.at[pl.ds(base, C)], didx)
        pltpu.sync_copy(w_hbm.at[pl.ds(base, C)], wv)
        pltpu.sync_copy(wv, deg_sh.at[didx], add=True)
        return carry

    lax.fori_loop(0, NCHUNK, body, 0)
    plsc.subcore_barrier()

    pltpu.sync_copy(deg_sh.at[pl.ds(s * ROWS_T, ROWS_T)], stage)
    pltpu.sync_copy(stage, degp_hbm.at[pl.ds(c * N_PAD + s * ROWS_T, ROWS_T)])


def _k2_body(x_ref, w1_ref, degp_ref, h_ref, dinv_ref):
    h_ref[...] = jnp.dot(x_ref[...], w1_ref[...], preferred_element_type=F32)
    deg = degp_ref[0, :] + degp_ref[1, :]
    dinv_ref[...] = jnp.where(deg > 0.0, lax.rsqrt(deg), 0.0)


def _k2(x, W1, degp):
    return pl.pallas_call(
        _k2_body,
        out_shape=(
            jax.ShapeDtypeStruct((N, DH), F32),
            jax.ShapeDtypeStruct((N_PAD,), F32),
        ),
    )(x, W1, degp)


@functools.partial(
    pl.kernel,
    out_type=jax.ShapeDtypeStruct((NC * N_PAD, DH), F32),
    mesh=_mesh,
    scratch_types=[
        pltpu.VMEM((N_PAD,), F32),
        pltpu.VMEM((C,), jnp.int32),
        pltpu.VMEM((C,), jnp.int32),
        pltpu.VMEM((C,), F32),
        pltpu.VMEM((C,), F32),
        pltpu.VMEM((C, DH), F32),
        pltpu.VMEM((ZB, DH), F32),
        pltpu.VMEM_SHARED((N_PAD, DH), F32),
        pltpu.SemaphoreType.DMA,
    ],
    compiler_params=pltpu.CompilerParams(
        needs_layout_passes=False, use_tc_tiling_on_sc=False
    ),
)
def _msg_kernel(src_hbm, dst_hbm, w_hbm, h_hbm, dinv_hbm, acc_hbm,
                dinv_v, sidx, didx, wv, tv, rows, zbuf, acc_sh, sem):
    c = lax.axis_index("c")
    s = lax.axis_index("s")

    pltpu.sync_copy(dinv_hbm, dinv_v)

    for j in range(ZB):
        for k in range(DH // 16):
            zbuf[j, pl.ds(k * 16, 16)] = jnp.zeros((16,), F32)

    def zero_body(i, carry):
        pltpu.sync_copy(zbuf, acc_sh.at[pl.ds(s * ROWS_T + i * ZB, ZB)])
        return carry

    lax.fori_loop(0, ROWS_T // ZB, zero_body, 0)
    plsc.subcore_barrier()

    tile_base = (c * NS + s) * ET

    def body(i, carry):
        base = tile_base + i * C
        pltpu.sync_copy(src_hbm.at[pl.ds(base, C)], sidx)
        pltpu.sync_copy(dst_hbm.at[pl.ds(base, C)], didx)
        pltpu.sync_copy(w_hbm.at[pl.ds(base, C)], wv)
        pltpu.async_copy(h_hbm.at[sidx], rows, sem).wait()

        for k in range(C // 16):
            sl = pl.ds(k * 16, 16)
            dv = plsc.load_gather(dinv_v, [sidx[sl]])
            tv[sl] = wv[sl] * dv

        def scale(i16, carry2):
            tvec = tv[pl.ds(i16 * 16, 16)]
            for l in range(16):
                tl = tvec[l]
                for k in range(DH // 16):
                    sl = pl.ds(k * 16, 16)
                    rows[i16 * 16 + l, sl] = rows[i16 * 16 + l, sl] * tl
            return carry2

        lax.fori_loop(0, C // 16, scale, 0)
        pltpu.sync_copy(rows, acc_sh.at[didx], add=True)
        return carry

    lax.fori_loop(0, NCHUNK, body, 0)
    plsc.subcore_barrier()

    def out_body(i, carry):
        row0 = s * ROWS_T + i * ZB
        pltpu.sync_copy(acc_sh.at[pl.ds(row0, ZB)], zbuf)

        def scale_out(i16, carry2):
            dvec = dinv_v[pl.ds(row0 + i16 * 16, 16)]
            for l in range(16):
                dl = dvec[l]
                for k in range(DH // 16):
                    sl = pl.ds(k * 16, 16)
                    zbuf[i16 * 16 + l, sl] = zbuf[i16 * 16 + l, sl] * dl
            return carry2

        lax.fori_loop(0, ZB // 16, scale_out, 0)
        pltpu.sync_copy(zbuf, acc_hbm.at[pl.ds(c * N_PAD + row0, ZB)])
        return carry

    lax.fori_loop(0, ROWS_T // ZB, out_body, 0)


def _k4_body(accp_ref, b1_ref, bexp_ref, wlin_ref, blin_ref, out_ref, pooled):
    a = accp_ref[0, 0:N, :] + accp_ref[1, 0:N, :]
    r = jnp.maximum(a + b1_ref[...], 0.0)
    bexp = bexp_ref[...]
    for g in range(G):
        v = jnp.where(bexp == g, r, 0.0)
        pooled[g, :] = jnp.max(v, axis=0)
    out_ref[...] = (
        jnp.dot(pooled[...], wlin_ref[...], preferred_element_type=F32)
        + blin_ref[...]
    )


def _k4(accp, b1r, bexp, Wlin, blinr):
    return pl.pallas_call(
        _k4_body,
        out_shape=jax.ShapeDtypeStruct((G, DOUT), F32),
        scratch_shapes=[pltpu.VMEM((G, DH), F32)],
    )(accp, b1r, bexp, Wlin, blinr)


def kernel(x, edge_index, edge_weight, batch, W1, b1, Wlin, blin):
    src, dst = edge_index[0], edge_index[1]
    loop = jnp.arange(N, dtype=jnp.int32)
    pad = E_PAD - E_RAW - N
    src_f = jnp.concatenate([src, loop, jnp.zeros((pad,), jnp.int32)])
    dst_f = jnp.concatenate([dst, loop, jnp.zeros((pad,), jnp.int32)])
    w_f = jnp.concatenate(
        [edge_weight, jnp.full((N,), 2.0, F32), jnp.zeros((pad,), F32)]
    )

    degp = _deg_kernel(dst_f, w_f)
    h, dinv = _k2(x, W1, degp.reshape(NC, N_PAD))
    accp = _msg_kernel(src_f, dst_f, w_f, h, dinv)
    bexp = jnp.broadcast_to(batch.astype(jnp.int32)[:, None], (N, DH))
    out = _k4(
        accp.reshape(NC, N_PAD, DH),
        b1.reshape(1, DH),
        bexp,
        Wlin,
        blin.reshape(1, DOUT),
    )
    return out


# speedup vs baseline: 19.9235x; 1.8826x over previous
import functools

import jax
import jax.numpy as jnp
from jax import lax
from jax.experimental import pallas as pl
from jax.experimental.pallas import tpu as pltpu
from jax.experimental.pallas import tpu_sc as plsc

N = 10000
DIN = 128
DH = 64
DOUT = 16
G = 32

NC = 2
NS = 16
NW = NC * NS
N_PAD = 10240
ROWS_T = N_PAD // NS

E_RAW = 320000
C = 128
NCHUNK = 82
ET = NCHUNK * C
E_PAD = ET * NW

ZB = 64
F32 = jnp.float32

_mesh = plsc.VectorSubcoreMesh(core_axis_name="c", subcore_axis_name="s")


@functools.partial(
    pl.kernel,
    out_type=jax.ShapeDtypeStruct((NC * N_PAD,), F32),
    mesh=_mesh,
    scratch_types=[
        pltpu.VMEM((ET,), jnp.int32),
        pltpu.VMEM((ET,), F32),
        pltpu.VMEM((C,), jnp.int32),
        pltpu.VMEM((C,), jnp.int32),
        pltpu.VMEM((C,), F32),
        pltpu.VMEM((ROWS_T,), F32),
        pltpu.VMEM_SHARED((N_PAD,), F32),
        pltpu.SemaphoreType.DMA,
        pltpu.SemaphoreType.DMA,
        pltpu.SemaphoreType.DMA,
    ],
)
def _deg_kernel(dst_hbm, w_hbm, degp_hbm, didx_all, w_all, db0, db1, dummy,
                stage, deg_sh, psem, s0, s1):
    c = lax.axis_index("c")
    s = lax.axis_index("s")
    ebase = (c * NS + s) * ET

    d1 = pltpu.async_copy(dst_hbm.at[pl.ds(ebase, ET)], didx_all, psem)
    d2 = pltpu.async_copy(w_hbm.at[pl.ds(ebase, ET)], w_all, psem)

    for i in range(ROWS_T // 16):
        stage[pl.ds(i * 16, 16)] = jnp.zeros((16,), F32)
    d1.wait()
    d2.wait()
    pltpu.sync_copy(stage, deg_sh.at[pl.ds(s * ROWS_T, ROWS_T)])
    plsc.subcore_barrier()

    def issue(i, buf, sem):
        for k in range(C // 16):
            buf[pl.ds(k * 16, 16)] = didx_all[pl.ds(i * C + k * 16, 16)]
        pltpu.async_copy(w_all.at[pl.ds(i * C, C)], deg_sh.at[buf], sem,
                         add=True)

    def drain(sem):
        pltpu.make_async_copy(w_hbm.at[pl.ds(0, C)], dummy, sem).wait()

    issue(0, db0, s0)
    issue(1, db1, s1)

    def pair(k2, carry):
        i = 2 * k2 + 2
        drain(s0)
        issue(i, db0, s0)
        drain(s1)
        issue(i + 1, db1, s1)
        return carry

    lax.fori_loop(0, (NCHUNK - 2) // 2, pair, 0)
    drain(s0)
    drain(s1)
    plsc.subcore_barrier()

    pltpu.sync_copy(deg_sh.at[pl.ds(s * ROWS_T, ROWS_T)], stage)
    pltpu.sync_copy(stage, degp_hbm.at[pl.ds(c * N_PAD + s * ROWS_T, ROWS_T)])


def _k2_body(x_ref, w1_ref, degp_ref, h_ref, dinv_ref):
    h_ref[...] = jnp.dot(x_ref[...], w1_ref[...], preferred_element_type=F32)
    deg = degp_ref[0, :] + degp_ref[1, :]
    dinv_ref[...] = jnp.where(deg > 0.0, lax.rsqrt(deg), 0.0)


def _k2(x, W1, degp):
    return pl.pallas_call(
        _k2_body,
        out_shape=(
            jax.ShapeDtypeStruct((N, DH), F32),
            jax.ShapeDtypeStruct((N_PAD,), F32),
        ),
    )(x, W1, degp)


@functools.partial(
    pl.kernel,
    out_type=jax.ShapeDtypeStruct((NC * N_PAD, DH), F32),
    mesh=_mesh,
    scratch_types=[
        pltpu.VMEM((N_PAD,), F32),
        pltpu.VMEM((ET,), jnp.int32),
        pltpu.VMEM((ET,), jnp.int32),
        pltpu.VMEM((ET,), F32),
        pltpu.VMEM((C,), F32),
        pltpu.VMEM((C, DH), F32),
        pltpu.VMEM((C, DH), F32),
        pltpu.VMEM((C,), jnp.int32),
        pltpu.VMEM((C,), jnp.int32),
        pltpu.VMEM((ZB, DH), F32),
        pltpu.VMEM_SHARED((N_PAD, DH), F32),
        pltpu.SemaphoreType.DMA,
        pltpu.SemaphoreType.DMA,
        pltpu.SemaphoreType.DMA,
        pltpu.SemaphoreType.DMA,
        pltpu.SemaphoreType.DMA,
    ],
    compiler_params=pltpu.CompilerParams(
        needs_layout_passes=False, use_tc_tiling_on_sc=False
    ),
)
def _msg_kernel(src_hbm, dst_hbm, w_hbm, h_hbm, dinv_hbm, acc_hbm,
                dinv_v, sidx_all, didx_all, w_all, tv, rows0, rows1,
                db0, db1, zbuf, acc_sh, psem, g0, g1, s0, s1):
    c = lax.axis_index("c")
    s = lax.axis_index("s")
    ebase = (c * NS + s) * ET

    d1 = pltpu.async_copy(src_hbm.at[pl.ds(ebase, ET)], sidx_all, psem)
    d2 = pltpu.async_copy(dst_hbm.at[pl.ds(ebase, ET)], didx_all, psem)
    d3 = pltpu.async_copy(w_hbm.at[pl.ds(ebase, ET)], w_all, psem)
    d4 = pltpu.async_copy(dinv_hbm, dinv_v, psem)

    for j in range(ZB):
        for k in range(DH // 16):
            zbuf[j, pl.ds(k * 16, 16)] = jnp.zeros((16,), F32)
    d1.wait()
    d2.wait()
    d3.wait()
    d4.wait()

    def zero_body(i, carry):
        pltpu.sync_copy(zbuf, acc_sh.at[pl.ds(s * ROWS_T + i * ZB, ZB)])
        return carry

    lax.fori_loop(0, ROWS_T // ZB, zero_body, 0)
    plsc.subcore_barrier()

    def start_gather(i, rows, gsem):
        pltpu.async_copy(h_hbm.at[sidx_all.at[pl.ds(i * C, C)]], rows, gsem)

    def wait_gather(rows, gsem):
        pltpu.make_async_copy(h_hbm.at[pl.ds(0, C)], rows, gsem).wait()

    def start_scatter(i, rows, db, ssem):
        for k in range(C // 16):
            db[pl.ds(k * 16, 16)] = didx_all[pl.ds(i * C + k * 16, 16)]
        pltpu.async_copy(rows, acc_sh.at[db], ssem, add=True)

    def wait_scatter(rows, ssem):
        pltpu.make_async_copy(h_hbm.at[pl.ds(0, C)], rows, ssem).wait()

    def scale(i, rows):
        for k in range(C // 16):
            sl = pl.ds(k * 16, 16)
            sv = sidx_all[pl.ds(i * C + k * 16, 16)]
            dv = plsc.load_gather(dinv_v, [sv])
            tv[sl] = w_all[pl.ds(i * C + k * 16, 16)] * dv

        def scale16(j16, carry2):
            tvec = tv[pl.ds(j16 * 16, 16)]
            for l in range(16):
                tl = tvec[l]
                for k in range(DH // 16):
                    sl = pl.ds(k * 16, 16)
                    rows[j16 * 16 + l, sl] = rows[j16 * 16 + l, sl] * tl
            return carry2

        lax.fori_loop(0, C // 16, scale16, 0)

    start_gather(0, rows0, g0)
    wait_gather(rows0, g0)
    start_gather(1, rows1, g1)
    scale(0, rows0)
    start_scatter(0, rows0, db0, s0)

    def step(i, rows, gsem, db, ssem, orows, ogsem, ossem):
        wait_gather(rows, gsem)
        wait_scatter(orows, ossem)
        start_gather(i + 1, orows, ogsem)
        scale(i, rows)
        start_scatter(i, rows, db, ssem)

    def pair(k2, carry):
        i = 2 * k2 + 1
        step(i, rows1, g1, db1, s1, rows0, g0, s0)
        step(i + 1, rows0, g0, db0, s0, rows1, g1, s1)
        return carry

    lax.fori_loop(0, (NCHUNK - 2) // 2, pair, 0)
    wait_gather(rows1, g1)
    scale(NCHUNK - 1, rows1)
    start_scatter(NCHUNK - 1, rows1, db1, s1)
    wait_scatter(rows0, s0)
    wait_scatter(rows1, s1)
    plsc.subcore_barrier()

    def out_body(i, carry):
        row0 = s * ROWS_T + i * ZB
        pltpu.sync_copy(acc_sh.at[pl.ds(row0, ZB)], zbuf)

        def scale_out(i16, carry2):
            dvec = dinv_v[pl.ds(row0 + i16 * 16, 16)]
            for l in range(16):
                dl = dvec[l]
                for k in range(DH // 16):
                    sl = pl.ds(k * 16, 16)
                    zbuf[i16 * 16 + l, sl] = zbuf[i16 * 16 + l, sl] * dl
            return carry2

        lax.fori_loop(0, ZB // 16, scale_out, 0)
        pltpu.sync_copy(zbuf, acc_hbm.at[pl.ds(c * N_PAD + row0, ZB)])
        return carry

    lax.fori_loop(0, ROWS_T // ZB, out_body, 0)


def _k4_body(accp_ref, b1_ref, bexp_ref, wlin_ref, blin_ref, out_ref, pooled):
    a = accp_ref[0, 0:N, :] + accp_ref[1, 0:N, :]
    r = jnp.maximum(a + b1_ref[...], 0.0)
    bexp = bexp_ref[...]
    for g in range(G):
        v = jnp.where(bexp == g, r, 0.0)
        pooled[g, :] = jnp.max(v, axis=0)
    out_ref[...] = (
        jnp.dot(pooled[...], wlin_ref[...], preferred_element_type=F32)
        + blin_ref[...]
    )


def _k4(accp, b1r, bexp, Wlin, blinr):
    return pl.pallas_call(
        _k4_body,
        out_shape=jax.ShapeDtypeStruct((G, DOUT), F32),
        scratch_shapes=[pltpu.VMEM((G, DH), F32)],
    )(accp, b1r, bexp, Wlin, blinr)


def kernel(x, edge_index, edge_weight, batch, W1, b1, Wlin, blin):
    src, dst = edge_index[0], edge_index[1]
    loop = jnp.arange(N, dtype=jnp.int32)
    pad = E_PAD - E_RAW - N
    src_f = jnp.concatenate([src, loop, jnp.zeros((pad,), jnp.int32)])
    dst_f = jnp.concatenate([dst, loop, jnp.zeros((pad,), jnp.int32)])
    w_f = jnp.concatenate(
        [edge_weight, jnp.full((N,), 2.0, F32), jnp.zeros((pad,), F32)]
    )

    degp = _deg_kernel(dst_f, w_f)
    h, dinv = _k2(x, W1, degp.reshape(NC, N_PAD))
    accp = _msg_kernel(src_f, dst_f, w_f, h, dinv)
    bexp = jnp.broadcast_to(batch.astype(jnp.int32)[:, None], (N, DH))
    out = _k4(
        accp.reshape(NC, N_PAD, DH),
        b1.reshape(1, DH),
        bexp,
        Wlin,
        blin.reshape(1, DOUT),
    )
    return out


# speedup vs baseline: 20.5444x; 1.0312x over previous
import functools

import jax
import jax.numpy as jnp
from jax import lax
from jax.experimental import pallas as pl
from jax.experimental.pallas import tpu as pltpu
from jax.experimental.pallas import tpu_sc as plsc

N = 10000
DIN = 128
DH = 64
DOUT = 16
G = 32

NC = 2
NS = 16
NW = NC * NS
N_PAD = 10240
ROWS_T = N_PAD // NS

E_RAW = 320000
C = 128
NCHUNK = 82
ET = NCHUNK * C
E_PAD = ET * NW

ZB = 64
F32 = jnp.float32

_mesh = plsc.VectorSubcoreMesh(core_axis_name="c", subcore_axis_name="s")


@functools.partial(
    pl.kernel,
    out_type=jax.ShapeDtypeStruct((NC * N_PAD,), F32),
    mesh=_mesh,
    scratch_types=[
        pltpu.VMEM((ET,), jnp.int32),
        pltpu.VMEM((ET,), F32),
        pltpu.VMEM((C,), jnp.int32),
        pltpu.VMEM((C,), jnp.int32),
        pltpu.VMEM((C,), F32),
        pltpu.VMEM((ROWS_T,), F32),
        pltpu.VMEM_SHARED((N_PAD,), F32),
        pltpu.SemaphoreType.DMA,
        pltpu.SemaphoreType.DMA,
        pltpu.SemaphoreType.DMA,
    ],
)
def _deg_kernel(dst_hbm, w_hbm, degp_hbm, didx_all, w_all, db0, db1, dummy,
                stage, deg_sh, psem, s0, s1):
    c = lax.axis_index("c")
    s = lax.axis_index("s")
    ebase = (c * NS + s) * ET

    d1 = pltpu.async_copy(dst_hbm.at[pl.ds(ebase, ET)], didx_all, psem)
    d2 = pltpu.async_copy(w_hbm.at[pl.ds(ebase, ET)], w_all, psem)

    for i in range(ROWS_T // 16):
        stage[pl.ds(i * 16, 16)] = jnp.zeros((16,), F32)
    d1.wait()
    d2.wait()
    pltpu.sync_copy(stage, deg_sh.at[pl.ds(s * ROWS_T, ROWS_T)])
    plsc.subcore_barrier()

    def issue(i, buf, sem):
        for k in range(C // 16):
            buf[pl.ds(k * 16, 16)] = didx_all[pl.ds(i * C + k * 16, 16)]
        pltpu.async_copy(w_all.at[pl.ds(i * C, C)], deg_sh.at[buf], sem,
                         add=True)

    def drain(sem):
        pltpu.make_async_copy(w_hbm.at[pl.ds(0, C)], dummy, sem).wait()

    issue(0, db0, s0)
    issue(1, db1, s1)

    def pair(k2, carry):
        i = 2 * k2 + 2
        drain(s0)
        issue(i, db0, s0)
        drain(s1)
        issue(i + 1, db1, s1)
        return carry

    lax.fori_loop(0, (NCHUNK - 2) // 2, pair, 0)
    drain(s0)
    drain(s1)
    plsc.subcore_barrier()

    pltpu.sync_copy(deg_sh.at[pl.ds(s * ROWS_T, ROWS_T)], stage)
    pltpu.sync_copy(stage, degp_hbm.at[pl.ds(c * N_PAD + s * ROWS_T, ROWS_T)])


def _k2_body(x_ref, w1_ref, degp_ref, h_ref, dinv_ref):
    h_ref[...] = jnp.dot(x_ref[...], w1_ref[...], preferred_element_type=F32)
    deg = degp_ref[0, :] + degp_ref[1, :]
    dinv_ref[...] = jnp.where(deg > 0.0, lax.rsqrt(deg), 0.0)


def _k2(x, W1, degp):
    return pl.pallas_call(
        _k2_body,
        out_shape=(
            jax.ShapeDtypeStruct((N, DH), F32),
            jax.ShapeDtypeStruct((N_PAD,), F32),
        ),
    )(x, W1, degp)


@functools.partial(
    pl.kernel,
    out_type=jax.ShapeDtypeStruct((NC * N_PAD, DH), F32),
    mesh=_mesh,
    scratch_types=[
        pltpu.VMEM((N_PAD,), F32),
        pltpu.VMEM((ET,), jnp.int32),
        pltpu.VMEM((ET,), jnp.int32),
        pltpu.VMEM((ET,), F32),
        pltpu.VMEM((C,), F32),
        pltpu.VMEM((C, DH), F32),
        pltpu.VMEM((C, DH), F32),
        pltpu.VMEM((C,), jnp.int32),
        pltpu.VMEM((C,), jnp.int32),
        pltpu.VMEM((ZB, DH), F32),
        pltpu.VMEM_SHARED((N_PAD, DH), F32),
        pltpu.SemaphoreType.DMA,
        pltpu.SemaphoreType.DMA,
        pltpu.SemaphoreType.DMA,
        pltpu.SemaphoreType.DMA,
        pltpu.SemaphoreType.DMA,
    ],
    compiler_params=pltpu.CompilerParams(
        needs_layout_passes=False, use_tc_tiling_on_sc=False
    ),
)
def _msg_kernel(src_hbm, dst_hbm, w_hbm, h_hbm, dinv_hbm, acc_hbm,
                dinv_v, sidx_all, didx_all, w_all, tv, rows0, rows1,
                db0, db1, zbuf, acc_sh, psem, g0, g1, s0, s1):
    c = lax.axis_index("c")
    s = lax.axis_index("s")
    ebase = (c * NS + s) * ET

    d1 = pltpu.async_copy(src_hbm.at[pl.ds(ebase, ET)], sidx_all, psem)
    d2 = pltpu.async_copy(dst_hbm.at[pl.ds(ebase, ET)], didx_all, psem)
    d3 = pltpu.async_copy(w_hbm.at[pl.ds(ebase, ET)], w_all, psem)
    d4 = pltpu.async_copy(dinv_hbm, dinv_v, psem)

    for j in range(ZB):
        for k in range(DH // 16):
            zbuf[j, pl.ds(k * 16, 16)] = jnp.zeros((16,), F32)
    d1.wait()
    d2.wait()
    d3.wait()
    d4.wait()

    def zero_body(i, carry):
        pltpu.sync_copy(zbuf, acc_sh.at[pl.ds(s * ROWS_T + i * ZB, ZB)])
        return carry

    lax.fori_loop(0, ROWS_T // ZB, zero_body, 0)
    plsc.subcore_barrier()

    def start_gather(i, rows, gsem):
        pltpu.async_copy(h_hbm.at[sidx_all.at[pl.ds(i * C, C)]], rows, gsem)

    def wait_gather(rows, gsem):
        pltpu.make_async_copy(h_hbm.at[pl.ds(0, C)], rows, gsem).wait()

    def start_scatter(i, rows, db, ssem):
        for k in range(C // 16):
            db[pl.ds(k * 16, 16)] = didx_all[pl.ds(i * C + k * 16, 16)]
        pltpu.async_copy(rows, acc_sh.at[db], ssem, add=True)

    def wait_scatter(rows, ssem):
        pltpu.make_async_copy(h_hbm.at[pl.ds(0, C)], rows, ssem).wait()

    def scale(i, rows):
        for k in range(C // 16):
            sl = pl.ds(k * 16, 16)
            sv = sidx_all[pl.ds(i * C + k * 16, 16)]
            dv = plsc.load_gather(dinv_v, [sv])
            tv[sl] = w_all[pl.ds(i * C + k * 16, 16)] * dv

        def scale16(j16, carry2):
            tvec = tv[pl.ds(j16 * 16, 16)]
            for l in range(16):
                tl = tvec[l]
                for k in range(DH // 16):
                    sl = pl.ds(k * 16, 16)
                    rows[j16 * 16 + l, sl] = rows[j16 * 16 + l, sl] * tl
            return carry2

        lax.fori_loop(0, C // 16, scale16, 0)

    start_gather(0, rows0, g0)
    wait_gather(rows0, g0)
    start_gather(1, rows1, g1)
    scale(0, rows0)
    start_scatter(0, rows0, db0, s0)

    def step(i, rows, gsem, db, ssem, orows, ogsem, ossem):
        wait_scatter(orows, ossem)
        start_gather(i + 1, orows, ogsem)
        wait_gather(rows, gsem)
        scale(i, rows)
        start_scatter(i, rows, db, ssem)

    def pair(k2, carry):
        i = 2 * k2 + 1
        step(i, rows1, g1, db1, s1, rows0, g0, s0)
        step(i + 1, rows0, g0, db0, s0, rows1, g1, s1)
        return carry

    lax.fori_loop(0, (NCHUNK - 2) // 2, pair, 0)
    wait_gather(rows1, g1)
    scale(NCHUNK - 1, rows1)
    start_scatter(NCHUNK - 1, rows1, db1, s1)
    wait_scatter(rows0, s0)
    wait_scatter(rows1, s1)
    plsc.subcore_barrier()

    def out_body(i, carry):
        row0 = s * ROWS_T + i * ZB
        pltpu.sync_copy(acc_sh.at[pl.ds(row0, ZB)], zbuf)

        def scale_out(i16, carry2):
            dvec = dinv_v[pl.ds(row0 + i16 * 16, 16)]
            for l in range(16):
                dl = dvec[l]
                for k in range(DH // 16):
                    sl = pl.ds(k * 16, 16)
                    zbuf[i16 * 16 + l, sl] = zbuf[i16 * 16 + l, sl] * dl
            return carry2

        lax.fori_loop(0, ZB // 16, scale_out, 0)
        pltpu.sync_copy(zbuf, acc_hbm.at[pl.ds(c * N_PAD + row0, ZB)])
        return carry

    lax.fori_loop(0, ROWS_T // ZB, out_body, 0)


def _k4_body(accp_ref, b1_ref, bexp_ref, wlin_ref, blin_ref, out_ref, pooled):
    a = accp_ref[0, 0:N, :] + accp_ref[1, 0:N, :]
    r = jnp.maximum(a + b1_ref[...], 0.0)
    bexp = bexp_ref[...]
    for g in range(G):
        v = jnp.where(bexp == g, r, 0.0)
        pooled[g, :] = jnp.max(v, axis=0)
    out_ref[...] = (
        jnp.dot(pooled[...], wlin_ref[...], preferred_element_type=F32)
        + blin_ref[...]
    )


def _k4(accp, b1r, bexp, Wlin, blinr):
    return pl.pallas_call(
        _k4_body,
        out_shape=jax.ShapeDtypeStruct((G, DOUT), F32),
        scratch_shapes=[pltpu.VMEM((G, DH), F32)],
    )(accp, b1r, bexp, Wlin, blinr)


def kernel(x, edge_index, edge_weight, batch, W1, b1, Wlin, blin):
    src, dst = edge_index[0], edge_index[1]
    loop = jnp.arange(N, dtype=jnp.int32)
    pad = E_PAD - E_RAW - N
    src_f = jnp.concatenate([src, loop, jnp.zeros((pad,), jnp.int32)])
    dst_f = jnp.concatenate([dst, loop, jnp.zeros((pad,), jnp.int32)])
    w_f = jnp.concatenate(
        [edge_weight, jnp.full((N,), 2.0, F32), jnp.zeros((pad,), F32)]
    )

    degp = _deg_kernel(dst_f, w_f)
    h, dinv = _k2(x, W1, degp.reshape(NC, N_PAD))
    accp = _msg_kernel(src_f, dst_f, w_f, h, dinv)
    bexp = jnp.broadcast_to(batch.astype(jnp.int32)[:, None], (N, DH))
    out = _k4(
        accp.reshape(NC, N_PAD, DH),
        b1.reshape(1, DH),
        bexp,
        Wlin,
        blin.reshape(1, DOUT),
    )
    return out


# speedup vs baseline: 24.8671x; 1.2104x over previous
import functools

import jax
import jax.numpy as jnp
from jax import lax
from jax.experimental import pallas as pl
from jax.experimental.pallas import tpu as pltpu
from jax.experimental.pallas import tpu_sc as plsc

N = 10000
DIN = 128
DH = 64
DOUT = 16
G = 32

NC = 2
NS = 16
NW = NC * NS
N_PAD = 10240
ROWS_T = N_PAD // NS

E_RAW = 320000
C = 128
NCHUNK = 82
ET = NCHUNK * C
E_PAD = ET * NW

ZB = 32
F32 = jnp.float32

_mesh = plsc.VectorSubcoreMesh(core_axis_name="c", subcore_axis_name="s")


@functools.partial(
    pl.kernel,
    out_type=jax.ShapeDtypeStruct((NC * N_PAD,), F32),
    mesh=_mesh,
    scratch_types=[
        pltpu.VMEM((ET,), jnp.int32),
        pltpu.VMEM((ET,), F32),
        pltpu.VMEM((C,), jnp.int32),
        pltpu.VMEM((C,), jnp.int32),
        pltpu.VMEM((C,), F32),
        pltpu.VMEM((ROWS_T,), F32),
        pltpu.VMEM_SHARED((N_PAD,), F32),
        pltpu.SemaphoreType.DMA,
        pltpu.SemaphoreType.DMA,
        pltpu.SemaphoreType.DMA,
    ],
)
def _deg_kernel(dst_hbm, w_hbm, degp_hbm, didx_all, w_all, db0, db1, dummy,
                stage, deg_sh, psem, s0, s1):
    c = lax.axis_index("c")
    s = lax.axis_index("s")
    ebase = (c * NS + s) * ET

    d1 = pltpu.async_copy(dst_hbm.at[pl.ds(ebase, ET)], didx_all, psem)
    d2 = pltpu.async_copy(w_hbm.at[pl.ds(ebase, ET)], w_all, psem)

    for i in range(ROWS_T // 16):
        stage[pl.ds(i * 16, 16)] = jnp.zeros((16,), F32)
    d1.wait()
    d2.wait()
    pltpu.sync_copy(stage, deg_sh.at[pl.ds(s * ROWS_T, ROWS_T)])
    plsc.subcore_barrier()

    def issue(i, buf, sem):
        for k in range(C // 16):
            buf[pl.ds(k * 16, 16)] = didx_all[pl.ds(i * C + k * 16, 16)]
        pltpu.async_copy(w_all.at[pl.ds(i * C, C)], deg_sh.at[buf], sem,
                         add=True)

    def drain(sem):
        pltpu.make_async_copy(w_hbm.at[pl.ds(0, C)], dummy, sem).wait()

    issue(0, db0, s0)
    issue(1, db1, s1)

    def pair(k2, carry):
        i = 2 * k2 + 2
        drain(s0)
        issue(i, db0, s0)
        drain(s1)
        issue(i + 1, db1, s1)
        return carry

    lax.fori_loop(0, (NCHUNK - 2) // 2, pair, 0)
    drain(s0)
    drain(s1)
    plsc.subcore_barrier()

    pltpu.sync_copy(deg_sh.at[pl.ds(s * ROWS_T, ROWS_T)], stage)
    pltpu.sync_copy(stage, degp_hbm.at[pl.ds(c * N_PAD + s * ROWS_T, ROWS_T)])


def _k2_body(x_ref, w1_ref, degp_ref, h_ref, dinv_ref):
    h = jnp.dot(x_ref[...], w1_ref[...], preferred_element_type=F32)
    h_ref[...] = h.astype(jnp.bfloat16)
    deg = degp_ref[0, :] + degp_ref[1, :]
    dinv_ref[...] = jnp.where(deg > 0.0, lax.rsqrt(deg), 0.0)


def _k2(x, W1, degp):
    return pl.pallas_call(
        _k2_body,
        out_shape=(
            jax.ShapeDtypeStruct((N_PAD, DH), jnp.bfloat16),
            jax.ShapeDtypeStruct((N_PAD,), F32),
        ),
    )(x, W1, degp)


@functools.partial(
    pl.kernel,
    out_type=jax.ShapeDtypeStruct((NC * N_PAD, DH), F32),
    mesh=_mesh,
    scratch_types=[
        pltpu.VMEM((N_PAD,), F32),
        pltpu.VMEM((ET,), jnp.int32),
        pltpu.VMEM((ET,), jnp.int32),
        pltpu.VMEM((ET,), F32),
        pltpu.VMEM((C,), F32),
        pltpu.VMEM((C, DH), jnp.bfloat16),
        pltpu.VMEM((C, DH), jnp.bfloat16),
        pltpu.VMEM((C, DH), F32),
        pltpu.VMEM((C, DH), F32),
        pltpu.VMEM((C,), jnp.int32),
        pltpu.VMEM((C,), jnp.int32),
        pltpu.VMEM((ZB, DH), F32),
        pltpu.VMEM_SHARED((N_PAD, DH), F32),
        pltpu.VMEM_SHARED((N_PAD, DH), jnp.bfloat16),
        pltpu.SemaphoreType.DMA,
        pltpu.SemaphoreType.DMA,
        pltpu.SemaphoreType.DMA,
        pltpu.SemaphoreType.DMA,
        pltpu.SemaphoreType.DMA,
    ],
    compiler_params=pltpu.CompilerParams(
        needs_layout_passes=False, use_tc_tiling_on_sc=False
    ),
)
def _msg_kernel(src_hbm, dst_hbm, w_hbm, h_hbm, dinv_hbm, acc_hbm,
                dinv_v, sidx_all, didx_all, w_all, tv, rows0, rows1,
                rowsf0, rowsf1, db0, db1, zbuf, acc_sh, h_sh,
                psem, g0, g1, s0, s1):
    c = lax.axis_index("c")
    s = lax.axis_index("s")
    ebase = (c * NS + s) * ET

    d1 = pltpu.async_copy(src_hbm.at[pl.ds(ebase, ET)], sidx_all, psem)
    d2 = pltpu.async_copy(dst_hbm.at[pl.ds(ebase, ET)], didx_all, psem)
    d3 = pltpu.async_copy(w_hbm.at[pl.ds(ebase, ET)], w_all, psem)
    d4 = pltpu.async_copy(dinv_hbm, dinv_v, psem)
    d5 = pltpu.async_copy(
        h_hbm.at[pl.ds(s * ROWS_T, ROWS_T)],
        h_sh.at[pl.ds(s * ROWS_T, ROWS_T)], psem)

    for j in range(ZB):
        for k in range(DH // 16):
            zbuf[j, pl.ds(k * 16, 16)] = jnp.zeros((16,), F32)
    d1.wait()
    d2.wait()
    d3.wait()
    d4.wait()
    d5.wait()

    def zero_body(i, carry):
        pltpu.sync_copy(zbuf, acc_sh.at[pl.ds(s * ROWS_T + i * ZB, ZB)])
        return carry

    lax.fori_loop(0, ROWS_T // ZB, zero_body, 0)
    plsc.subcore_barrier()

    def start_gather(i, rows, gsem):
        pltpu.async_copy(h_sh.at[sidx_all.at[pl.ds(i * C, C)]], rows, gsem)

    def wait_gather(rows, gsem):
        pltpu.make_async_copy(h_hbm.at[pl.ds(0, C)], rows, gsem).wait()

    def start_scatter(i, rows, db, ssem):
        for k in range(C // 16):
            db[pl.ds(k * 16, 16)] = didx_all[pl.ds(i * C + k * 16, 16)]
        pltpu.async_copy(rows, acc_sh.at[db], ssem, add=True)

    def wait_scatter(rowsf, ssem):
        pltpu.make_async_copy(acc_hbm.at[pl.ds(0, C)], rowsf, ssem).wait()

    def scale(i, rows, rowsf):
        for k in range(C // 16):
            sl = pl.ds(k * 16, 16)
            sv = sidx_all[pl.ds(i * C + k * 16, 16)]
            dv = plsc.load_gather(dinv_v, [sv])
            tv[sl] = w_all[pl.ds(i * C + k * 16, 16)] * dv

        def scale16(j16, carry2):
            tvec = tv[pl.ds(j16 * 16, 16)]
            for l in range(16):
                tl = tvec[l]
                j = j16 * 16 + l
                for k in range(DH // 32):
                    v = rows[j, pl.ds(k * 32, 32)]
                    a, b = plsc.unpack(v, format=plsc.PackFormat.INTERLEAVED)
                    rowsf[j, pl.ds(k * 32, 16)] = a * tl
                    rowsf[j, pl.ds(k * 32 + 16, 16)] = b * tl
            return carry2

        lax.fori_loop(0, C // 16, scale16, 0)

    start_gather(0, rows0, g0)
    wait_gather(rows0, g0)
    start_gather(1, rows1, g1)
    scale(0, rows0, rowsf0)
    start_scatter(0, rowsf0, db0, s0)

    def step(i, rows, gsem, rowsf, db, ssem, orows, ogsem, orowsf, ossem):
        wait_scatter(orowsf, ossem)
        start_gather(i + 1, orows, ogsem)
        wait_gather(rows, gsem)
        scale(i, rows, rowsf)
        start_scatter(i, rowsf, db, ssem)

    def pair(k2, carry):
        i = 2 * k2 + 1
        step(i, rows1, g1, rowsf1, db1, s1, rows0, g0, rowsf0, s0)
        step(i + 1, rows0, g0, rowsf0, db0, s0, rows1, g1, rowsf1, s1)
        return carry

    lax.fori_loop(0, (NCHUNK - 2) // 2, pair, 0)
    wait_gather(rows1, g1)
    scale(NCHUNK - 1, rows1, rowsf1)
    start_scatter(NCHUNK - 1, rowsf1, db1, s1)
    wait_scatter(rowsf0, s0)
    wait_scatter(rowsf1, s1)
    plsc.subcore_barrier()

    def out_body(i, carry):
        row0 = s * ROWS_T + i * ZB
        pltpu.sync_copy(acc_sh.at[pl.ds(row0, ZB)], zbuf)

        def scale_out(i16, carry2):
            dvec = dinv_v[pl.ds(row0 + i16 * 16, 16)]
            for l in range(16):
                dl = dvec[l]
                for k in range(DH // 16):
                    sl = pl.ds(k * 16, 16)
                    zbuf[i16 * 16 + l, sl] = zbuf[i16 * 16 + l, sl] * dl
            return carry2

        lax.fori_loop(0, ZB // 16, scale_out, 0)
        pltpu.sync_copy(zbuf, acc_hbm.at[pl.ds(c * N_PAD + row0, ZB)])
        return carry

    lax.fori_loop(0, ROWS_T // ZB, out_body, 0)


def _k4_body(accp_ref, b1_ref, bexp_ref, wlin_ref, blin_ref, out_ref, pooled):
    a = accp_ref[0, 0:N, :] + accp_ref[1, 0:N, :]
    r = jnp.maximum(a + b1_ref[...], 0.0)
    bexp = bexp_ref[...]
    for g in range(G):
        v = jnp.where(bexp == g, r, 0.0)
        pooled[g, :] = jnp.max(v, axis=0)
    out_ref[...] = (
        jnp.dot(pooled[...], wlin_ref[...], preferred_element_type=F32)
        + blin_ref[...]
    )


def _k4(accp, b1r, bexp, Wlin, blinr):
    return pl.pallas_call(
        _k4_body,
        out_shape=jax.ShapeDtypeStruct((G, DOUT), F32),
        scratch_shapes=[pltpu.VMEM((G, DH), F32)],
    )(accp, b1r, bexp, Wlin, blinr)


def kernel(x, edge_index, edge_weight, batch, W1, b1, Wlin, blin):
    src, dst = edge_index[0], edge_index[1]
    loop = jnp.arange(N, dtype=jnp.int32)
    pad = E_PAD - E_RAW - N
    src_f = jnp.concatenate([src, loop, jnp.zeros((pad,), jnp.int32)])
    dst_f = jnp.concatenate([dst, loop, jnp.zeros((pad,), jnp.int32)])
    w_f = jnp.concatenate(
        [edge_weight, jnp.full((N,), 2.0, F32), jnp.zeros((pad,), F32)]
    )

    degp = _deg_kernel(dst_f, w_f)
    x_p = jnp.concatenate([x, jnp.zeros((N_PAD - N, DIN), F32)])
    sigma = [(j // 32) * 32 + (j % 2) * 16 + (j % 32) // 2 for j in range(DH)]
    h, dinv = _k2(x_p, W1[:, jnp.array(sigma)], degp.reshape(NC, N_PAD))
    accp = _msg_kernel(src_f, dst_f, w_f, h, dinv)
    bexp = jnp.broadcast_to(batch.astype(jnp.int32)[:, None], (N, DH))
    out = _k4(
        accp.reshape(NC, N_PAD, DH),
        b1.reshape(1, DH),
        bexp,
        Wlin,
        blin.reshape(1, DOUT),
    )
    return out


# speedup vs baseline: 25.0616x; 1.0078x over previous
import functools

import jax
import jax.numpy as jnp
from jax import lax
from jax.experimental import pallas as pl
from jax.experimental.pallas import tpu as pltpu
from jax.experimental.pallas import tpu_sc as plsc

N = 10000
DIN = 128
DH = 64
DOUT = 16
G = 32

NC = 2
NS = 16
NW = NC * NS
N_PAD = 10240
ROWS_T = N_PAD // NS

E_RAW = 320000
C = 128
NCHUNK = 82
ET = NCHUNK * C
E_PAD = ET * NW

ZB = 32
F32 = jnp.float32

_mesh = plsc.VectorSubcoreMesh(core_axis_name="c", subcore_axis_name="s")


@functools.partial(
    pl.kernel,
    out_type=jax.ShapeDtypeStruct((NC * N_PAD,), F32),
    mesh=_mesh,
    scratch_types=[
        pltpu.VMEM((ET,), jnp.int32),
        pltpu.VMEM((ET,), F32),
        pltpu.VMEM((C,), jnp.int32),
        pltpu.VMEM((C,), jnp.int32),
        pltpu.VMEM((C,), F32),
        pltpu.VMEM((ROWS_T,), F32),
        pltpu.VMEM_SHARED((N_PAD,), F32),
        pltpu.SemaphoreType.DMA,
        pltpu.SemaphoreType.DMA,
        pltpu.SemaphoreType.DMA,
    ],
)
def _deg_kernel(dst_hbm, w_hbm, degp_hbm, didx_all, w_all, db0, db1, dummy,
                stage, deg_sh, psem, s0, s1):
    c = lax.axis_index("c")
    s = lax.axis_index("s")
    ebase = (c * NS + s) * ET

    d1 = pltpu.async_copy(dst_hbm.at[pl.ds(ebase, ET)], didx_all, psem)
    d2 = pltpu.async_copy(w_hbm.at[pl.ds(ebase, ET)], w_all, psem)

    for i in range(ROWS_T // 16):
        stage[pl.ds(i * 16, 16)] = jnp.zeros((16,), F32)
    d1.wait()
    d2.wait()
    pltpu.sync_copy(stage, deg_sh.at[pl.ds(s * ROWS_T, ROWS_T)])
    plsc.subcore_barrier()

    def issue(i, buf, sem):
        for k in range(C // 16):
            buf[pl.ds(k * 16, 16)] = didx_all[pl.ds(i * C + k * 16, 16)]
        pltpu.async_copy(w_all.at[pl.ds(i * C, C)], deg_sh.at[buf], sem,
                         add=True)

    def drain(sem):
        pltpu.make_async_copy(w_hbm.at[pl.ds(0, C)], dummy, sem).wait()

    issue(0, db0, s0)
    issue(1, db1, s1)

    def pair(k2, carry):
        i = 2 * k2 + 2
        drain(s0)
        issue(i, db0, s0)
        drain(s1)
        issue(i + 1, db1, s1)
        return carry

    lax.fori_loop(0, (NCHUNK - 2) // 2, pair, 0)
    drain(s0)
    drain(s1)
    plsc.subcore_barrier()

    pltpu.sync_copy(deg_sh.at[pl.ds(s * ROWS_T, ROWS_T)], stage)
    pltpu.sync_copy(stage, degp_hbm.at[pl.ds(c * N_PAD + s * ROWS_T, ROWS_T)])


def _k2_body(x_ref, w1_ref, h_ref):
    h = jnp.dot(x_ref[...], w1_ref[...], preferred_element_type=F32)
    h_ref[...] = h.astype(jnp.bfloat16)


def _k2(x, W1):
    return pl.pallas_call(
        _k2_body,
        out_shape=jax.ShapeDtypeStruct((N_PAD, DH), jnp.bfloat16),
    )(x, W1)


@functools.partial(
    pl.kernel,
    out_type=jax.ShapeDtypeStruct((NC * N_PAD, DH), F32),
    mesh=_mesh,
    scratch_types=[
        pltpu.VMEM((N_PAD,), F32),
        pltpu.VMEM((ET,), jnp.int32),
        pltpu.VMEM((ET,), jnp.int32),
        pltpu.VMEM((ET,), F32),
        pltpu.VMEM((C,), F32),
        pltpu.VMEM((C, DH), jnp.bfloat16),
        pltpu.VMEM((C, DH), jnp.bfloat16),
        pltpu.VMEM((C, DH), F32),
        pltpu.VMEM((C, DH), F32),
        pltpu.VMEM((C,), jnp.int32),
        pltpu.VMEM((C,), jnp.int32),
        pltpu.VMEM((ZB, DH), F32),
        pltpu.VMEM_SHARED((N_PAD, DH), F32),
        pltpu.VMEM_SHARED((N_PAD, DH), jnp.bfloat16),
        pltpu.VMEM_SHARED((N_PAD,), F32),
        pltpu.SemaphoreType.DMA,
        pltpu.SemaphoreType.DMA,
        pltpu.SemaphoreType.DMA,
        pltpu.SemaphoreType.DMA,
        pltpu.SemaphoreType.DMA,
    ],
    compiler_params=pltpu.CompilerParams(
        needs_layout_passes=False, use_tc_tiling_on_sc=False
    ),
)
def _msg_kernel(src_hbm, dst_hbm, w_hbm, h_hbm, degp_hbm, acc_hbm,
                dinv_v, sidx_all, didx_all, w_all, tv, rows0, rows1,
                rowsf0, rowsf1, db0, db1, zbuf, acc_sh, h_sh, deg_sh,
                psem, g0, g1, s0, s1):
    c = lax.axis_index("c")
    s = lax.axis_index("s")
    ebase = (c * NS + s) * ET

    d1 = pltpu.async_copy(src_hbm.at[pl.ds(ebase, ET)], sidx_all, psem)
    d2 = pltpu.async_copy(dst_hbm.at[pl.ds(ebase, ET)], didx_all, psem)
    d3 = pltpu.async_copy(w_hbm.at[pl.ds(ebase, ET)], w_all, psem)
    d5 = pltpu.async_copy(
        h_hbm.at[pl.ds(s * ROWS_T, ROWS_T)],
        h_sh.at[pl.ds(s * ROWS_T, ROWS_T)], psem)
    d6 = pltpu.async_copy(degp_hbm.at[pl.ds(s * ROWS_T, ROWS_T)],
                          dinv_v.at[pl.ds(0, ROWS_T)], psem)
    d7 = pltpu.async_copy(degp_hbm.at[pl.ds(N_PAD + s * ROWS_T, ROWS_T)],
                          dinv_v.at[pl.ds(ROWS_T, ROWS_T)], psem)

    for j in range(ZB):
        for k in range(DH // 16):
            zbuf[j, pl.ds(k * 16, 16)] = jnp.zeros((16,), F32)
    d1.wait()
    d2.wait()
    d3.wait()
    d5.wait()
    d6.wait()
    d7.wait()

    def zero_body(i, carry):
        pltpu.sync_copy(zbuf, acc_sh.at[pl.ds(s * ROWS_T + i * ZB, ZB)])
        return carry

    lax.fori_loop(0, ROWS_T // ZB, zero_body, 0)
    plsc.subcore_barrier()

    for k in range(ROWS_T // 16):
        sl = pl.ds(k * 16, 16)
        xdeg = dinv_v[sl] + dinv_v[pl.ds(ROWS_T + k * 16, 16)]
        ibits = plsc.bitcast(xdeg, jnp.int32)
        y = plsc.bitcast(
            jnp.full((16,), 0x5F3759DF, jnp.int32)
            - lax.shift_right_logical(ibits, jnp.full((16,), 1, jnp.int32)),
            F32,
        )
        for _ in range(4):
            y = y * (1.5 - 0.5 * xdeg * y * y)
        dinv_v[sl] = y
    pltpu.sync_copy(dinv_v.at[pl.ds(0, ROWS_T)],
                    deg_sh.at[pl.ds(s * ROWS_T, ROWS_T)])
    plsc.subcore_barrier()
    pltpu.sync_copy(deg_sh, dinv_v)


    def start_gather(i, rows, gsem):
        pltpu.async_copy(h_sh.at[sidx_all.at[pl.ds(i * C, C)]], rows, gsem)

    def wait_gather(rows, gsem):
        pltpu.make_async_copy(h_hbm.at[pl.ds(0, C)], rows, gsem).wait()

    def start_scatter(i, rows, db, ssem):
        for k in range(C // 16):
            db[pl.ds(k * 16, 16)] = didx_all[pl.ds(i * C + k * 16, 16)]
        pltpu.async_copy(rows, acc_sh.at[db], ssem, add=True)

    def wait_scatter(rowsf, ssem):
        pltpu.make_async_copy(acc_hbm.at[pl.ds(0, C)], rowsf, ssem).wait()

    def scale(i, rows, rowsf):
        for k in range(C // 16):
            sl = pl.ds(k * 16, 16)
            sv = sidx_all[pl.ds(i * C + k * 16, 16)]
            dv = plsc.load_gather(dinv_v, [sv])
            tv[sl] = w_all[pl.ds(i * C + k * 16, 16)] * dv

        def scale16(j16, carry2):
            tvec = tv[pl.ds(j16 * 16, 16)]
            for l in range(16):
                tl = tvec[l]
                j = j16 * 16 + l
                for k in range(DH // 32):
                    v = rows[j, pl.ds(k * 32, 32)]
                    a, b = plsc.unpack(v, format=plsc.PackFormat.INTERLEAVED)
                    rowsf[j, pl.ds(k * 32, 16)] = a * tl
                    rowsf[j, pl.ds(k * 32 + 16, 16)] = b * tl
            return carry2

        lax.fori_loop(0, C // 16, scale16, 0)

    start_gather(0, rows0, g0)
    wait_gather(rows0, g0)
    start_gather(1, rows1, g1)
    scale(0, rows0, rowsf0)
    start_scatter(0, rowsf0, db0, s0)

    def step(i, rows, gsem, rowsf, db, ssem, orows, ogsem, orowsf, ossem):
        wait_scatter(orowsf, ossem)
        start_gather(i + 1, orows, ogsem)
        wait_gather(rows, gsem)
        scale(i, rows, rowsf)
        start_scatter(i, rowsf, db, ssem)

    def pair(k2, carry):
        i = 2 * k2 + 1
        step(i, rows1, g1, rowsf1, db1, s1, rows0, g0, rowsf0, s0)
        step(i + 1, rows0, g0, rowsf0, db0, s0, rows1, g1, rowsf1, s1)
        return carry

    lax.fori_loop(0, (NCHUNK - 2) // 2, pair, 0)
    wait_gather(rows1, g1)
    scale(NCHUNK - 1, rows1, rowsf1)
    start_scatter(NCHUNK - 1, rowsf1, db1, s1)
    wait_scatter(rowsf0, s0)
    wait_scatter(rowsf1, s1)
    plsc.subcore_barrier()

    def out_body(i, carry):
        row0 = s * ROWS_T + i * ZB
        pltpu.sync_copy(acc_sh.at[pl.ds(row0, ZB)], zbuf)

        def scale_out(i16, carry2):
            dvec = dinv_v[pl.ds(row0 + i16 * 16, 16)]
            for l in range(16):
                dl = dvec[l]
                for k in range(DH // 16):
                    sl = pl.ds(k * 16, 16)
                    zbuf[i16 * 16 + l, sl] = zbuf[i16 * 16 + l, sl] * dl
            return carry2

        lax.fori_loop(0, ZB // 16, scale_out, 0)
        pltpu.sync_copy(zbuf, acc_hbm.at[pl.ds(c * N_PAD + row0, ZB)])
        return carry

    lax.fori_loop(0, ROWS_T // ZB, out_body, 0)


def _k4_body(accp_ref, b1_ref, bexp_ref, wlin_ref, blin_ref, out_ref, pooled):
    a = accp_ref[0, 0:N, :] + accp_ref[1, 0:N, :]
    r = jnp.maximum(a + b1_ref[...], 0.0)
    bexp = bexp_ref[...]
    for g in range(G):
        v = jnp.where(bexp == g, r, 0.0)
        pooled[g, :] = jnp.max(v, axis=0)
    out_ref[...] = (
        jnp.dot(pooled[...], wlin_ref[...], preferred_element_type=F32)
        + blin_ref[...]
    )


def _k4(accp, b1r, bexp, Wlin, blinr):
    return pl.pallas_call(
        _k4_body,
        out_shape=jax.ShapeDtypeStruct((G, DOUT), F32),
        scratch_shapes=[pltpu.VMEM((G, DH), F32)],
    )(accp, b1r, bexp, Wlin, blinr)


def kernel(x, edge_index, edge_weight, batch, W1, b1, Wlin, blin):
    src, dst = edge_index[0], edge_index[1]
    loop = jnp.arange(N, dtype=jnp.int32)
    pad = E_PAD - E_RAW - N
    src_f = jnp.concatenate([src, loop, jnp.zeros((pad,), jnp.int32)])
    dst_f = jnp.concatenate([dst, loop, jnp.zeros((pad,), jnp.int32)])
    w_f = jnp.concatenate(
        [edge_weight, jnp.full((N,), 2.0, F32), jnp.zeros((pad,), F32)]
    )

    x_p = jnp.concatenate([x, jnp.zeros((N_PAD - N, DIN), F32)])
    sigma = [(j // 32) * 32 + (j % 2) * 16 + (j % 32) // 2 for j in range(DH)]
    h = _k2(x_p, W1[:, jnp.array(sigma)])
    degp = _deg_kernel(dst_f, w_f)
    accp = _msg_kernel(src_f, dst_f, w_f, h, degp)
    bexp = jnp.broadcast_to(batch.astype(jnp.int32)[:, None], (N, DH))
    out = _k4(
        accp.reshape(NC, N_PAD, DH),
        b1.reshape(1, DH),
        bexp,
        Wlin,
        blin.reshape(1, DOUT),
    )
    return out
